# Initial kernel scaffold; baseline (speedup 1.0000x reference)
import jax
import jax.numpy as jnp
from jax.experimental import pallas as pl


def kernel(x, edge_index, edge_attr, Wz, bz, Lz, lbz, Wr, br, Lr, lbr, Wh, bh, Lh, lbh, att, linW, linb):
    raise NotImplementedError("write your pallas kernel here")


# speedup vs baseline: 287.5409x; 287.5409x over previous
import functools

import jax
import jax.numpy as jnp
from jax import lax
from jax.experimental import pallas as pl
from jax.experimental.pallas import tpu as pltpu
from jax.experimental.pallas import tpu_sc as plsc

N = 10000
E = 320000
HID = 128
PERIODS = 12

NUM_CORES = 2
NUM_SUBCORES = 16
NUM_TILES = NUM_CORES * NUM_SUBCORES
LANES = 16

EDGES_PER_TILE = E // NUM_TILES
AGG_CHUNK = 16000

@functools.cache
def _sc_mesh():
  return plsc.VectorSubcoreMesh(
      core_axis_name="c", subcore_axis_name="s",
      num_cores=NUM_CORES, num_subcores=NUM_SUBCORES)


def _zero_vmem(ref, n):
  zero = jnp.zeros((LANES,), jnp.float32)

  def body(i, _):
    ref[pl.ds(i * LANES, LANES)] = zero
    return 0

  lax.fori_loop(0, n // LANES, body, 0)


@functools.cache
def _sc_deg_kernel():
  return pl.kernel(
      _sc_deg_body,
      out_type=jax.ShapeDtypeStruct((NUM_TILES, N), jnp.float32),
      mesh=_sc_mesh(),
      compiler_params=pltpu.CompilerParams(needs_layout_passes=False),
      scratch_types=[
          pltpu.VMEM((EDGES_PER_TILE,), jnp.int32),
          pltpu.VMEM((EDGES_PER_TILE,), jnp.float32),
          pltpu.VMEM((N,), jnp.float32),
      ],
  )


def _sc_deg_body(dst_hbm, w_hbm, out_hbm, dst_v, w_v, deg_v):
  c = lax.axis_index("c")
  s = lax.axis_index("s")
  wid = s * NUM_CORES + c
  base = wid * EDGES_PER_TILE
  pltpu.sync_copy(dst_hbm.at[pl.ds(base, EDGES_PER_TILE)], dst_v)
  pltpu.sync_copy(w_hbm.at[pl.ds(base, EDGES_PER_TILE)], w_v)
  _zero_vmem(deg_v, N)

  def ebody(i, _):
    sl = pl.ds(i * LANES, LANES)
    plsc.addupdate_scatter(deg_v, [dst_v[sl]], w_v[sl])
    return 0

  lax.fori_loop(0, EDGES_PER_TILE // LANES, ebody, 0)
  pltpu.sync_copy(deg_v, out_hbm.at[wid])


def _tc_prep_body(degp_ref, xT_ref, qT_ref, disT_ref):
  deg = jnp.sum(degp_ref[...], axis=0, keepdims=True) + 1.0
  dis = lax.rsqrt(deg)
  disT_ref[...] = dis
  qT_ref[...] = xT_ref[...] * dis


def _tc_prep(degp, xT):
  return pl.pallas_call(
      _tc_prep_body,
      out_shape=[
          jax.ShapeDtypeStruct((PERIODS, N), jnp.float32),
          jax.ShapeDtypeStruct((1, N), jnp.float32),
      ],
  )(degp, xT)


_HALF = E // NUM_CORES


@functools.cache
def _sc_agg_kernel():
  return pl.kernel(
      _sc_agg_body,
      out_type=jax.ShapeDtypeStruct((PERIODS, NUM_CORES, N), jnp.float32),
      mesh=_sc_mesh(),
      compiler_params=pltpu.CompilerParams(needs_layout_passes=False),
      scratch_types=[
          pltpu.VMEM((N,), jnp.float32),
          pltpu.VMEM((N,), jnp.float32),
          pltpu.VMEM((AGG_CHUNK,), jnp.int32),
          pltpu.VMEM((AGG_CHUNK,), jnp.int32),
          pltpu.VMEM((AGG_CHUNK,), jnp.float32),
      ],
  )


def _sc_agg_body(qT_hbm, src_hbm, dst_hbm, w_hbm, out_hbm, q_v, acc_v, src_v,
                 dst_v, w_v):
  c = lax.axis_index("c")
  s = lax.axis_index("s")

  @pl.when(s < PERIODS)
  def _():
    pltpu.sync_copy(qT_hbm.at[s], q_v)
    _zero_vmem(acc_v, N)
    base0 = c * _HALF

    def chunk(j, _):
      base = base0 + j * AGG_CHUNK
      pltpu.sync_copy(src_hbm.at[pl.ds(base, AGG_CHUNK)], src_v)
      pltpu.sync_copy(dst_hbm.at[pl.ds(base, AGG_CHUNK)], dst_v)
      pltpu.sync_copy(w_hbm.at[pl.ds(base, AGG_CHUNK)], w_v)

      def ebody(i, _):
        sl = pl.ds(i * LANES, LANES)
        qv = plsc.load_gather(q_v, [src_v[sl]])
        plsc.addupdate_scatter(acc_v, [dst_v[sl]], w_v[sl] * qv)
        return 0

      lax.fori_loop(0, AGG_CHUNK // LANES, ebody, 0)
      return 0

    lax.fori_loop(0, _HALF // AGG_CHUNK, chunk, 0)
    pltpu.sync_copy(acc_v, out_hbm.at[s, c])


_GRU_BLK = 1024
_NP = 10240


def _tc_gru_body(T3_ref, qT_ref, disT_ref, Wz_ref, Lz_ref, lbz_ref, Wr_ref,
                 Lr_ref, lbr_ref, Wh_ref, Lh_ref, lbh_ref, bz_ref, br_ref,
                 bh_ref, att_ref, linW_ref, linb_ref, y_ref):
  f32 = jnp.float32
  Ts = T3_ref[:, 0, :] + T3_ref[:, 1, :]
  aggT = disT_ref[...] * (Ts + qT_ref[...])

  att = att_ref[...]
  e = jnp.exp(att - jnp.max(att))
  probs = e / jnp.sum(e)

  def fold(W_ref, L_ref, b_ref, lb_ref):
    L = L_ref[...]
    Ltop, U = L[:HID], L[HID:]
    v = jnp.dot(W_ref[...], Ltop, preferred_element_type=f32)
    cst = jnp.dot(b_ref[...], Ltop, preferred_element_type=f32) + lb_ref[...]
    return v, cst, U

  vz, cz, Uz = fold(Wz_ref, Lz_ref, bz_ref, lbz_ref)
  vr, cr, Ur = fold(Wr_ref, Lr_ref, br_ref, lbr_ref)
  vh, ch, Uh = fold(Wh_ref, Lh_ref, bh_ref, lbh_ref)

  H = jnp.zeros((_GRU_BLK, HID), f32)
  Hacc = jnp.zeros((_GRU_BLK, HID), f32)
  for p in range(PERIODS):
    sel = (lax.broadcasted_iota(jnp.int32, (PERIODS, 1), 0) == p).astype(f32)
    acol = lax.dot_general(aggT, sel, (((0,), (0,)), ((), ())),
                           preferred_element_type=f32)
    Z = jax.nn.sigmoid(acol * vz
                       + jnp.dot(H, Uz, preferred_element_type=f32) + cz)
    R = jax.nn.sigmoid(acol * vr
                       + jnp.dot(H, Ur, preferred_element_type=f32) + cr)
    Ht = jnp.tanh(acol * vh
                  + jnp.dot(H * R, Uh, preferred_element_type=f32) + ch)
    H = Z * H + (1.0 - Z) * Ht
    Hacc = Hacc + probs[0, p] * H

  y_ref[...] = (jnp.dot(jnp.maximum(Hacc, 0.0), linW_ref[...],
                        preferred_element_type=f32) + linb_ref[...])


def _tc_gru(T3, qT, disT, Wz, Lz, lbz, Wr, Lr, lbr, Wh, Lh, lbh, bz, br, bh,
            att, linW, linb):
  full = lambda shape: pl.BlockSpec(shape, lambda i: tuple(0 for _ in shape))
  return pl.pallas_call(
      _tc_gru_body,
      grid=(_NP // _GRU_BLK,),
      in_specs=[
          pl.BlockSpec((PERIODS, NUM_CORES, _GRU_BLK), lambda i: (0, 0, i)),
          pl.BlockSpec((PERIODS, _GRU_BLK), lambda i: (0, i)),
          pl.BlockSpec((1, _GRU_BLK), lambda i: (0, i)),
          full((1, HID)), full((2 * HID, HID)), full((1, HID)),
          full((1, HID)), full((2 * HID, HID)), full((1, HID)),
          full((1, HID)), full((2 * HID, HID)), full((1, HID)),
          full((1, HID)), full((1, HID)), full((1, HID)),
          full((1, PERIODS)), full((HID, 1)), full((1, 1)),
      ],
      out_specs=pl.BlockSpec((_GRU_BLK, 1), lambda i: (i, 0)),
      out_shape=jax.ShapeDtypeStruct((_NP, 1), jnp.float32),
  )(T3, qT, disT, Wz, Lz, lbz, Wr, Lr, lbr, Wh, Lh, lbh, bz, br, bh, att,
    linW, linb)


def kernel(x, edge_index, edge_attr, Wz, bz, Lz, lbz, Wr, br, Lr, lbr, Wh,
           bh, Lh, lbh, att, linW, linb):
  src = edge_index[0]
  dst = edge_index[1]
  xT = jnp.transpose(x[:, 0, :])

  degp = _sc_deg_kernel()(dst, edge_attr)
  qT, disT = _tc_prep(degp, xT)
  T3 = _sc_agg_kernel()(qT, src, dst, edge_attr)
  pad = _NP - N
  T3p = jnp.pad(T3, ((0, 0), (0, 0), (0, pad)))
  qTp = jnp.pad(qT, ((0, 0), (0, pad)))
  disTp = jnp.pad(disT, ((0, 0), (0, pad)), constant_values=1.0)
  y = _tc_gru(
      T3p, qTp, disTp,
      Wz, Lz, lbz.reshape(1, HID),
      Wr, Lr, lbr.reshape(1, HID),
      Wh, Lh, lbh.reshape(1, HID),
      bz.reshape(1, HID), br.reshape(1, HID), bh.reshape(1, HID),
      att.reshape(1, PERIODS), linW, linb.reshape(1, 1))
  return y[:N].reshape(-1)


# speedup vs baseline: 401.8328x; 1.3975x over previous
import functools

import jax
import jax.numpy as jnp
from jax import lax
from jax.experimental import pallas as pl
from jax.experimental.pallas import tpu as pltpu
from jax.experimental.pallas import tpu_sc as plsc

N = 10000
E = 320000
HID = 128
PERIODS = 12

NUM_CORES = 2
NUM_SUBCORES = 16
NUM_TILES = NUM_CORES * NUM_SUBCORES
LANES = 16

EDGES_PER_TILE = E // NUM_TILES
AGG_CHUNK = 16000

@functools.cache
def _sc_mesh():
  return plsc.VectorSubcoreMesh(
      core_axis_name="c", subcore_axis_name="s",
      num_cores=NUM_CORES, num_subcores=NUM_SUBCORES)


def _zero_vmem(ref, n):
  zero = jnp.zeros((LANES,), jnp.float32)

  @plsc.parallel_loop(0, n, step=LANES, unroll=8)
  def _(i):
    ref[pl.ds(i, LANES)] = zero


@functools.cache
def _sc_deg_kernel():
  return pl.kernel(
      _sc_deg_body,
      out_type=jax.ShapeDtypeStruct((NUM_TILES, N), jnp.float32),
      mesh=_sc_mesh(),
      compiler_params=pltpu.CompilerParams(needs_layout_passes=False),
      scratch_types=[
          pltpu.VMEM((EDGES_PER_TILE,), jnp.int32),
          pltpu.VMEM((EDGES_PER_TILE,), jnp.float32),
          pltpu.VMEM((N,), jnp.float32),
      ],
  )


def _sc_deg_body(dst_hbm, w_hbm, out_hbm, dst_v, w_v, deg_v):
  c = lax.axis_index("c")
  s = lax.axis_index("s")
  wid = s * NUM_CORES + c
  base = wid * EDGES_PER_TILE
  pltpu.sync_copy(dst_hbm.at[pl.ds(base, EDGES_PER_TILE)], dst_v)
  pltpu.sync_copy(w_hbm.at[pl.ds(base, EDGES_PER_TILE)], w_v)
  _zero_vmem(deg_v, N)

  @plsc.parallel_loop(0, EDGES_PER_TILE, step=LANES, unroll=8)
  def _(i):
    sl = pl.ds(i, LANES)
    plsc.addupdate_scatter(deg_v, [dst_v[sl]], w_v[sl])

  pltpu.sync_copy(deg_v, out_hbm.at[wid])


def _tc_prep_body(degp_ref, xT_ref, qT_ref, disT_ref):
  deg = jnp.sum(degp_ref[...], axis=0, keepdims=True) + 1.0
  dis = lax.rsqrt(deg)
  disT_ref[...] = dis
  qT_ref[...] = xT_ref[...] * dis


def _tc_prep(degp, xT):
  return pl.pallas_call(
      _tc_prep_body,
      out_shape=[
          jax.ShapeDtypeStruct((PERIODS, N), jnp.float32),
          jax.ShapeDtypeStruct((1, N), jnp.float32),
      ],
  )(degp, xT)


_HALF = E // NUM_CORES


@functools.cache
def _sc_agg_kernel():
  return pl.kernel(
      _sc_agg_body,
      out_type=jax.ShapeDtypeStruct((PERIODS, NUM_CORES, N), jnp.float32),
      mesh=_sc_mesh(),
      compiler_params=pltpu.CompilerParams(needs_layout_passes=False),
      scratch_types=[
          pltpu.VMEM((N,), jnp.float32),
          pltpu.VMEM((N,), jnp.float32),
          pltpu.VMEM((AGG_CHUNK,), jnp.int32),
          pltpu.VMEM((AGG_CHUNK,), jnp.int32),
          pltpu.VMEM((AGG_CHUNK,), jnp.float32),
      ],
  )


def _sc_agg_body(qT_hbm, src_hbm, dst_hbm, w_hbm, out_hbm, q_v, acc_v, src_v,
                 dst_v, w_v):
  c = lax.axis_index("c")
  s = lax.axis_index("s")

  @pl.when(s < PERIODS)
  def _():
    pltpu.sync_copy(qT_hbm.at[s], q_v)
    _zero_vmem(acc_v, N)
    base0 = c * _HALF

    def chunk(j, _):
      base = base0 + j * AGG_CHUNK
      pltpu.sync_copy(src_hbm.at[pl.ds(base, AGG_CHUNK)], src_v)
      pltpu.sync_copy(dst_hbm.at[pl.ds(base, AGG_CHUNK)], dst_v)
      pltpu.sync_copy(w_hbm.at[pl.ds(base, AGG_CHUNK)], w_v)

      @plsc.parallel_loop(0, AGG_CHUNK, step=LANES, unroll=8)
      def _(i):
        sl = pl.ds(i, LANES)
        qv = plsc.load_gather(q_v, [src_v[sl]])
        plsc.addupdate_scatter(acc_v, [dst_v[sl]], w_v[sl] * qv)

      return 0

    lax.fori_loop(0, _HALF // AGG_CHUNK, chunk, 0)
    pltpu.sync_copy(acc_v, out_hbm.at[s, c])


_GRU_BLK = 1024
_NP = 10240


def _tc_gru_body(T3_ref, qT_ref, disT_ref, Wz_ref, Lz_ref, lbz_ref, Wr_ref,
                 Lr_ref, lbr_ref, Wh_ref, Lh_ref, lbh_ref, bz_ref, br_ref,
                 bh_ref, att_ref, linW_ref, linb_ref, y_ref):
  f32 = jnp.float32
  Ts = T3_ref[:, 0, :] + T3_ref[:, 1, :]
  aggT = disT_ref[...] * (Ts + qT_ref[...])

  att = att_ref[...]
  e = jnp.exp(att - jnp.max(att))
  probs = e / jnp.sum(e)

  def fold(W_ref, L_ref, b_ref, lb_ref):
    L = L_ref[...]
    Ltop, U = L[:HID], L[HID:]
    v = jnp.dot(W_ref[...], Ltop, preferred_element_type=f32)
    cst = jnp.dot(b_ref[...], Ltop, preferred_element_type=f32) + lb_ref[...]
    return v, cst, U

  vz, cz, Uz = fold(Wz_ref, Lz_ref, bz_ref, lbz_ref)
  vr, cr, Ur = fold(Wr_ref, Lr_ref, br_ref, lbr_ref)
  vh, ch, Uh = fold(Wh_ref, Lh_ref, bh_ref, lbh_ref)

  H = jnp.zeros((_GRU_BLK, HID), f32)
  Hacc = jnp.zeros((_GRU_BLK, HID), f32)
  for p in range(PERIODS):
    sel = (lax.broadcasted_iota(jnp.int32, (PERIODS, 1), 0) == p).astype(f32)
    acol = lax.dot_general(aggT, sel, (((0,), (0,)), ((), ())),
                           preferred_element_type=f32)
    Z = jax.nn.sigmoid(acol * vz
                       + jnp.dot(H, Uz, preferred_element_type=f32) + cz)
    R = jax.nn.sigmoid(acol * vr
                       + jnp.dot(H, Ur, preferred_element_type=f32) + cr)
    Ht = jnp.tanh(acol * vh
                  + jnp.dot(H * R, Uh, preferred_element_type=f32) + ch)
    H = Z * H + (1.0 - Z) * Ht
    Hacc = Hacc + probs[0, p] * H

  y_ref[...] = (jnp.dot(jnp.maximum(Hacc, 0.0), linW_ref[...],
                        preferred_element_type=f32) + linb_ref[...])


def _tc_gru(T3, qT, disT, Wz, Lz, lbz, Wr, Lr, lbr, Wh, Lh, lbh, bz, br, bh,
            att, linW, linb):
  full = lambda shape: pl.BlockSpec(shape, lambda i: tuple(0 for _ in shape))
  return pl.pallas_call(
      _tc_gru_body,
      grid=(_NP // _GRU_BLK,),
      in_specs=[
          pl.BlockSpec((PERIODS, NUM_CORES, _GRU_BLK), lambda i: (0, 0, i)),
          pl.BlockSpec((PERIODS, _GRU_BLK), lambda i: (0, i)),
          pl.BlockSpec((1, _GRU_BLK), lambda i: (0, i)),
          full((1, HID)), full((2 * HID, HID)), full((1, HID)),
          full((1, HID)), full((2 * HID, HID)), full((1, HID)),
          full((1, HID)), full((2 * HID, HID)), full((1, HID)),
          full((1, HID)), full((1, HID)), full((1, HID)),
          full((1, PERIODS)), full((HID, 1)), full((1, 1)),
      ],
      out_specs=pl.BlockSpec((_GRU_BLK, 1), lambda i: (i, 0)),
      out_shape=jax.ShapeDtypeStruct((_NP, 1), jnp.float32),
  )(T3, qT, disT, Wz, Lz, lbz, Wr, Lr, lbr, Wh, Lh, lbh, bz, br, bh, att,
    linW, linb)


def kernel(x, edge_index, edge_attr, Wz, bz, Lz, lbz, Wr, br, Lr, lbr, Wh,
           bh, Lh, lbh, att, linW, linb):
  src = edge_index[0]
  dst = edge_index[1]
  xT = jnp.transpose(x[:, 0, :])

  degp = _sc_deg_kernel()(dst, edge_attr)
  qT, disT = _tc_prep(degp, xT)
  T3 = _sc_agg_kernel()(qT, src, dst, edge_attr)
  pad = _NP - N
  T3p = jnp.pad(T3, ((0, 0), (0, 0), (0, pad)))
  qTp = jnp.pad(qT, ((0, 0), (0, pad)))
  disTp = jnp.pad(disT, ((0, 0), (0, pad)), constant_values=1.0)
  y = _tc_gru(
      T3p, qTp, disTp,
      Wz, Lz, lbz.reshape(1, HID),
      Wr, Lr, lbr.reshape(1, HID),
      Wh, Lh, lbh.reshape(1, HID),
      bz.reshape(1, HID), br.reshape(1, HID), bh.reshape(1, HID),
      att.reshape(1, PERIODS), linW, linb.reshape(1, 1))
  return y[:N].reshape(-1)


# speedup vs baseline: 423.5904x; 1.0541x over previous
import functools

import jax
import jax.numpy as jnp
from jax import lax
from jax.experimental import pallas as pl
from jax.experimental.pallas import tpu as pltpu
from jax.experimental.pallas import tpu_sc as plsc

N = 10000
E = 320000
HID = 128
PERIODS = 12

NUM_CORES = 2
NUM_SUBCORES = 16
NUM_TILES = NUM_CORES * NUM_SUBCORES
LANES = 16

EDGES_PER_TILE = E // NUM_TILES
AGG_CHUNK = 16000

@functools.cache
def _sc_mesh():
  return plsc.VectorSubcoreMesh(
      core_axis_name="c", subcore_axis_name="s",
      num_cores=NUM_CORES, num_subcores=NUM_SUBCORES)


def _zero_vmem(ref, n):
  zero = jnp.zeros((LANES,), jnp.float32)

  @plsc.parallel_loop(0, n, step=LANES, unroll=8)
  def _(i):
    ref[pl.ds(i, LANES)] = zero


@functools.cache
def _sc_deg_kernel():
  return pl.kernel(
      _sc_deg_body,
      out_type=jax.ShapeDtypeStruct((NUM_TILES, N), jnp.float32),
      mesh=_sc_mesh(),
      compiler_params=pltpu.CompilerParams(needs_layout_passes=False),
      scratch_types=[
          pltpu.VMEM((EDGES_PER_TILE,), jnp.int32),
          pltpu.VMEM((EDGES_PER_TILE,), jnp.float32),
          pltpu.VMEM((N,), jnp.float32),
      ],
  )


def _sc_deg_body(dst_hbm, w_hbm, out_hbm, dst_v, w_v, deg_v):
  c = lax.axis_index("c")
  s = lax.axis_index("s")
  wid = s * NUM_CORES + c
  base = wid * EDGES_PER_TILE
  pltpu.sync_copy(dst_hbm.at[pl.ds(base, EDGES_PER_TILE)], dst_v)
  pltpu.sync_copy(w_hbm.at[pl.ds(base, EDGES_PER_TILE)], w_v)
  _zero_vmem(deg_v, N)

  @plsc.parallel_loop(0, EDGES_PER_TILE, step=LANES, unroll=8)
  def _(i):
    sl = pl.ds(i, LANES)
    plsc.addupdate_scatter(deg_v, [dst_v[sl]], w_v[sl])

  pltpu.sync_copy(deg_v, out_hbm.at[wid])


def _tc_prep_body(degp_ref, xT_ref, qT_ref, disT_ref):
  deg = jnp.sum(degp_ref[...], axis=0, keepdims=True) + 1.0
  dis = lax.rsqrt(deg)
  disT_ref[...] = dis
  qT_ref[...] = xT_ref[...] * dis


def _tc_prep(degp, xT):
  return pl.pallas_call(
      _tc_prep_body,
      out_shape=[
          jax.ShapeDtypeStruct((PERIODS, N), jnp.float32),
          jax.ShapeDtypeStruct((1, N), jnp.float32),
      ],
  )(degp, xT)


_HALF = E // NUM_CORES


@functools.cache
def _sc_agg_kernel():
  return pl.kernel(
      _sc_agg_body,
      out_type=jax.ShapeDtypeStruct((PERIODS, NUM_CORES, N), jnp.float32),
      mesh=_sc_mesh(),
      compiler_params=pltpu.CompilerParams(needs_layout_passes=False),
      scratch_types=[
          pltpu.VMEM((N,), jnp.float32),
          pltpu.VMEM((N,), jnp.float32),
          pltpu.VMEM((AGG_CHUNK,), jnp.int32),
          pltpu.VMEM((AGG_CHUNK,), jnp.int32),
          pltpu.VMEM((AGG_CHUNK,), jnp.float32),
      ],
  )


def _sc_agg_body(qT_hbm, src_hbm, dst_hbm, w_hbm, out_hbm, q_v, acc_v, src_v,
                 dst_v, w_v):
  c = lax.axis_index("c")
  s = lax.axis_index("s")

  @pl.when(s < PERIODS)
  def _():
    pltpu.sync_copy(qT_hbm.at[s], q_v)
    _zero_vmem(acc_v, N)
    base0 = c * _HALF

    def chunk(j, _):
      base = base0 + j * AGG_CHUNK
      pltpu.sync_copy(src_hbm.at[pl.ds(base, AGG_CHUNK)], src_v)
      pltpu.sync_copy(dst_hbm.at[pl.ds(base, AGG_CHUNK)], dst_v)
      pltpu.sync_copy(w_hbm.at[pl.ds(base, AGG_CHUNK)], w_v)

      @plsc.parallel_loop(0, AGG_CHUNK, step=LANES, unroll=8)
      def _(i):
        sl = pl.ds(i, LANES)
        qv = plsc.load_gather(q_v, [src_v[sl]])
        plsc.addupdate_scatter(acc_v, [dst_v[sl]], w_v[sl] * qv)

      return 0

    lax.fori_loop(0, _HALF // AGG_CHUNK, chunk, 0)
    pltpu.sync_copy(acc_v, out_hbm.at[s, c])


_GRU_BLK = 2048


def _tc_gru_body(T3_ref, qT_ref, disT_ref, Wz_ref, Lz_ref, lbz_ref, Wr_ref,
                 Lr_ref, lbr_ref, Wh_ref, Lh_ref, lbh_ref, bz_ref, br_ref,
                 bh_ref, att_ref, linW_ref, linb_ref, y_ref):
  f32 = jnp.float32
  Ts = T3_ref[:, 0, :] + T3_ref[:, 1, :]
  aggT = disT_ref[...] * (Ts + qT_ref[...])

  att = att_ref[...]
  e = jnp.exp(att - jnp.max(att))
  probs = e / jnp.sum(e)

  def fold(W_ref, L_ref, b_ref, lb_ref):
    L = L_ref[...]
    Ltop, U = L[:HID], L[HID:]
    v = jnp.dot(W_ref[...], Ltop, preferred_element_type=f32)
    cst = jnp.dot(b_ref[...], Ltop, preferred_element_type=f32) + lb_ref[...]
    return v, cst, U

  vz, cz, Uz = fold(Wz_ref, Lz_ref, bz_ref, lbz_ref)
  vr, cr, Ur = fold(Wr_ref, Lr_ref, br_ref, lbr_ref)
  vh, ch, Uh = fold(Wh_ref, Lh_ref, bh_ref, lbh_ref)
  Uzr = jnp.concatenate([Uz, Ur], axis=1)
  vzr = jnp.concatenate([vz, vr], axis=1)
  czr = jnp.concatenate([cz, cr], axis=1)

  eye = (lax.broadcasted_iota(jnp.int32, (PERIODS, PERIODS), 0)
         == lax.broadcasted_iota(jnp.int32, (PERIODS, PERIODS), 1)).astype(f32)
  aggC = lax.dot_general(aggT, eye, (((0,), (0,)), ((), ())),
                         preferred_element_type=f32)

  H = jnp.zeros((_GRU_BLK, HID), f32)
  Hacc = jnp.zeros((_GRU_BLK, HID), f32)
  for p in range(PERIODS):
    acol = aggC[:, p:p + 1]
    ZR = jax.nn.sigmoid(acol * vzr
                        + jnp.dot(H, Uzr, preferred_element_type=f32) + czr)
    Z, R = ZR[:, :HID], ZR[:, HID:]
    Ht = jnp.tanh(acol * vh
                  + jnp.dot(H * R, Uh, preferred_element_type=f32) + ch)
    H = Z * H + (1.0 - Z) * Ht
    Hacc = Hacc + probs[0, p] * H

  y_ref[...] = (jnp.dot(jnp.maximum(Hacc, 0.0), linW_ref[...],
                        preferred_element_type=f32) + linb_ref[...])


def _tc_gru(T3, qT, disT, Wz, Lz, lbz, Wr, Lr, lbr, Wh, Lh, lbh, bz, br, bh,
            att, linW, linb):
  full = lambda shape: pl.BlockSpec(shape, lambda i: tuple(0 for _ in shape))
  return pl.pallas_call(
      _tc_gru_body,
      grid=(pl.cdiv(N, _GRU_BLK),),
      in_specs=[
          pl.BlockSpec((PERIODS, NUM_CORES, _GRU_BLK), lambda i: (0, 0, i)),
          pl.BlockSpec((PERIODS, _GRU_BLK), lambda i: (0, i)),
          pl.BlockSpec((1, _GRU_BLK), lambda i: (0, i)),
          full((1, HID)), full((2 * HID, HID)), full((1, HID)),
          full((1, HID)), full((2 * HID, HID)), full((1, HID)),
          full((1, HID)), full((2 * HID, HID)), full((1, HID)),
          full((1, HID)), full((1, HID)), full((1, HID)),
          full((1, PERIODS)), full((HID, 1)), full((1, 1)),
      ],
      out_specs=pl.BlockSpec((_GRU_BLK, 1), lambda i: (i, 0)),
      out_shape=jax.ShapeDtypeStruct((N, 1), jnp.float32),
  )(T3, qT, disT, Wz, Lz, lbz, Wr, Lr, lbr, Wh, Lh, lbh, bz, br, bh, att,
    linW, linb)


def kernel(x, edge_index, edge_attr, Wz, bz, Lz, lbz, Wr, br, Lr, lbr, Wh,
           bh, Lh, lbh, att, linW, linb):
  src = edge_index[0]
  dst = edge_index[1]
  xT = jnp.transpose(x[:, 0, :])

  degp = _sc_deg_kernel()(dst, edge_attr)
  qT, disT = _tc_prep(degp, xT)
  T3 = _sc_agg_kernel()(qT, src, dst, edge_attr)
  y = _tc_gru(
      T3, qT, disT,
      Wz, Lz, lbz.reshape(1, HID),
      Wr, Lr, lbr.reshape(1, HID),
      Wh, Lh, lbh.reshape(1, HID),
      bz.reshape(1, HID), br.reshape(1, HID), bh.reshape(1, HID),
      att.reshape(1, PERIODS), linW, linb.reshape(1, 1))
  return y.reshape(-1)
